# baseline (speedup 1.0000x reference)
import functools

import jax
import jax.numpy as jnp
from jax import lax
from jax.experimental import pallas as pl
from jax.experimental.pallas import tpu as pltpu
from jax.experimental.pallas import tpu_sc as plsc

_N = 10000
_E = 320000
_ALPHA = 0.8
_BN_EPS = 0.001
_PYG_BN_EPS = 1e-05


_BLK = 2000
_GRID = _N // _BLK

_P = jax.lax.Precision.HIGHEST


def _full_spec(arr):
  nd = arr.ndim
  return pl.BlockSpec(arr.shape, lambda i, _nd=nd: (0,) * _nd)


def _row_spec(cols):
  return pl.BlockSpec((_BLK, cols), lambda i: (i, 0))


def _elu(x):
  return jnp.where(x > 0, x, jnp.exp(x) - 1.0)


def _sigmoid(x):
  return 1.0 / (1.0 + jnp.exp(-x))


def _bn(x, g, b, eps):
  return x / jnp.sqrt(1.0 + eps) * g[None, :] + b[None, :]


def _tc1_body(x_ref, We0_ref, be0_ref, ge0_ref, bbe0_ref,
              We1_ref, be1_ref, ge1_ref, bbe1_ref,
              Wk_ref, bk_ref, Wqv_ref, bqv_ref, Ws_ref, bs_ref,
              f_ref, k_ref, qv_ref, s_ref):
  x = x_ref[...]
  h = _elu(_bn(x @ We0_ref[...] + be0_ref[...][None, :],
               ge0_ref[...], bbe0_ref[...], _BN_EPS))
  f = _elu(_bn(h @ We1_ref[...] + be1_ref[...][None, :],
               ge1_ref[...], bbe1_ref[...], _BN_EPS))
  f_ref[...] = f
  k_ref[...] = f @ Wk_ref[...] + bk_ref[...][None, :]
  qv_ref[...] = f @ Wqv_ref[...] + bqv_ref[...][None, :]
  s_ref[...] = f @ Ws_ref[...] + bs_ref[...][None, :]


def _tc2_body(a0_ref, a1_ref, s1_ref, gc_ref, bc_ref,
              Wk_ref, bk_ref, Wqv_ref, bqv_ref, Ws_ref, bs_ref,
              k_ref, qv_ref, s_ref):
  c1 = a0_ref[...] + a1_ref[...] + s1_ref[...]
  c1 = jnp.maximum(_bn(c1, gc_ref[...], bc_ref[...], _PYG_BN_EPS), 0.0)
  k_ref[...] = c1 @ Wk_ref[...] + bk_ref[...][None, :]
  qv_ref[...] = c1 @ Wqv_ref[...] + bqv_ref[...][None, :]
  s_ref[...] = c1 @ Ws_ref[...] + bs_ref[...][None, :]


def _tc3_body(f_ref, a0_ref, a1_ref, s2_ref,
              Wd0_ref, bd0_ref, gd0_ref, bbd0_ref,
              Wd1_ref, bd1_ref, gd1_ref, bbd1_ref,
              Wd2_ref, bd2_ref, gd2_ref, bbd2_ref,
              cl_ref,
              z_ref, mu_ref, lv_ref, de_ref, q_ref):
  t = a0_ref[...] + a1_ref[...] + s2_ref[...]
  mu = t[:, :8]
  lv = t[:, 8:]
  mu_ref[...] = mu
  lv_ref[...] = lv
  z = jnp.concatenate([f_ref[...], mu], axis=1)
  z_ref[...] = z
  d = _elu(_bn(z @ Wd0_ref[...] + bd0_ref[...][None, :],
               gd0_ref[...], bbd0_ref[...], _BN_EPS))
  d = _elu(_bn(d @ Wd1_ref[...] + bd1_ref[...][None, :],
               gd1_ref[...], bbd1_ref[...], _BN_EPS))
  de_ref[...] = _sigmoid(_bn(d @ Wd2_ref[...] + bd2_ref[...][None, :],
                             gd2_ref[...], bbd2_ref[...], _BN_EPS))
  cl = cl_ref[...]
  zn2 = jnp.sum(z * z, axis=1, keepdims=True)
  cn2 = jnp.sum(cl * cl, axis=1)[None, :]
  cross = jax.lax.dot_general(z, cl, (((1,), (1,)), ((), ())))
  dist2 = zn2 - 2.0 * cross + cn2
  base = 1.0 + dist2 / _ALPHA + 1e-08
  qq = jnp.exp((-(_ALPHA + 1.0) / 2.0) * jnp.log(base))
  q_ref[...] = qq / jnp.sum(qq, axis=1, keepdims=True)


_NC = 2
_NS = 16
_NW = _NC * _NS
_B = 128
_CHUNKS = _E // _B
_BASE = _CHUNKS // _NW
_EXTRA = _CHUNKS - _BASE * _NW
_NP = 10240
_RPT = _NP // _NS
_ZR = 128


@functools.lru_cache(maxsize=None)
def _make_edge_kernel(F, FQV):
  mesh = plsc.VectorSubcoreMesh(core_axis_name="c", subcore_axis_name="s",
                                num_cores=_NC, num_subcores=_NS)
  nf = F // 16

  @functools.partial(
      pl.kernel,
      out_type=jax.ShapeDtypeStruct((_NC * _NP, F), jnp.float32),
      mesh=mesh,
      compiler_params=pltpu.CompilerParams(use_tc_tiling_on_sc=False),
      scratch_types=[
          pltpu.VMEM((_BASE, _B), jnp.int32),
          pltpu.VMEM((_BASE, _B), jnp.int32),
          pltpu.VMEM((_B, F), jnp.float32),
          pltpu.VMEM((_B, F), jnp.float32),
          pltpu.VMEM((_B, FQV), jnp.float32),
          pltpu.VMEM((_B, FQV), jnp.float32),
          pltpu.VMEM((_B, F), jnp.float32),
          pltpu.VMEM((_ZR, F), jnp.float32),
          pltpu.VMEM_SHARED((_NP, F), jnp.float32),
          pltpu.SemaphoreType.DMA,
          pltpu.SemaphoreType.DMA,
      ],
  )
  def edge_kernel(k_hbm, qv_hbm, src_hbm, dst_hbm, out_hbm,
                  srcall, dstall, krows0, krows1, qvrows0, qvrows1,
                  msg, zbuf, acc, sem0, sem1):
    cid = lax.axis_index("c")
    sid = lax.axis_index("s")
    w = cid * _NS + sid
    kbufs = (krows0, krows1)
    qbufs = (qvrows0, qvrows1)
    sems = (sem0, sem1)

    def zrow(j, carry):
      for f0 in range(nf):
        zbuf[j, pl.ds(f0 * 16, 16)] = jnp.zeros((16,), jnp.float32)
      return carry

    lax.fori_loop(0, _ZR, zrow, 0)
    for t in range(_RPT // _ZR):
      pltpu.sync_copy(zbuf, acc.at[pl.ds(sid * _RPT + t * _ZR, _ZR)])
    plsc.subcore_barrier()

    pltpu.sync_copy(src_hbm.at[pl.ds(w * _BASE, _BASE)], srcall)
    pltpu.sync_copy(dst_hbm.at[pl.ds(w * _BASE, _BASE)], dstall)

    def start_gathers(j, b):
      pltpu.async_copy(k_hbm.at[dstall.at[j]], kbufs[b], sems[b])
      pltpu.async_copy(qv_hbm.at[srcall.at[j]], qbufs[b], sems[b])

    def wait_gathers(j, b):
      pltpu.make_async_copy(k_hbm.at[dstall.at[j]], kbufs[b], sems[b]).wait()
      pltpu.make_async_copy(qv_hbm.at[srcall.at[j]], qbufs[b], sems[b]).wait()

    def compute_scatter(j, b):
      kb = kbufs[b]
      qb = qbufs[b]

      @plsc.parallel_loop(0, _B, unroll=8)
      def row(r):
        for f0 in range(nf):
          kv = kb[r, pl.ds(f0 * 16, 16)]
          qv = qb[r, pl.ds(f0 * 16, 16)]
          vv = qb[r, pl.ds(F + f0 * 16, 16)]
          gate = 1.0 / (1.0 + jnp.exp(-(kv + qv)))
          msg[r, pl.ds(f0 * 16, 16)] = gate * vv

      pltpu.sync_copy(msg, acc.at[dstall.at[j]], add=True)

    start_gathers(0, 0)

    def body(j2, carry):
      for b in range(2):
        j = 2 * j2 + b

        @pl.when(j + 1 < _BASE)
        def _():
          start_gathers(j + 1, 1 - b)

        wait_gathers(j, b)
        compute_scatter(j, b)
      return carry

    lax.fori_loop(0, _BASE // 2, body, 0)

    @pl.when(w < _EXTRA)
    def _():
      pltpu.sync_copy(src_hbm.at[pl.ds(_NW * _BASE + w, 1)],
                      srcall.at[pl.ds(0, 1)])
      pltpu.sync_copy(dst_hbm.at[pl.ds(_NW * _BASE + w, 1)],
                      dstall.at[pl.ds(0, 1)])
      start_gathers(0, 0)
      wait_gathers(0, 0)
      compute_scatter(0, 0)

    plsc.subcore_barrier()
    pltpu.sync_copy(acc.at[pl.ds(sid * _RPT, _RPT)],
                    out_hbm.at[pl.ds(cid * _NP + sid * _RPT, _RPT)])

  return edge_kernel


@functools.lru_cache(maxsize=None)
def _make_edge_kernel_bf16_64():
  F = 64
  mesh = plsc.VectorSubcoreMesh(core_axis_name="c", subcore_axis_name="s",
                                num_cores=_NC, num_subcores=_NS)

  @functools.partial(
      pl.kernel,
      out_type=jax.ShapeDtypeStruct((_NC * _NP, F), jnp.float32),
      mesh=mesh,
      compiler_params=pltpu.CompilerParams(use_tc_tiling_on_sc=False,
                                           needs_layout_passes=False),
      scratch_types=[
          pltpu.VMEM((_BASE, _B), jnp.int32),
          pltpu.VMEM((_BASE, _B), jnp.int32),
          pltpu.VMEM((_B, F), jnp.bfloat16),
          pltpu.VMEM((_B, F), jnp.bfloat16),
          pltpu.VMEM((_B, F), jnp.bfloat16),
          pltpu.VMEM((_B, F), jnp.bfloat16),
          pltpu.VMEM((_B, F), jnp.float32),
          pltpu.VMEM((_B, F), jnp.float32),
          pltpu.VMEM((_B, F), jnp.float32),
          pltpu.VMEM((_ZR, F), jnp.float32),
          pltpu.VMEM_SHARED((_NP, F), jnp.float32),
          pltpu.SemaphoreType.DMA,
          pltpu.SemaphoreType.DMA,
      ],
  )
  def edge_kernel(k_hbm, q_hbm, v_hbm, src_hbm, dst_hbm, out_hbm,
                  srcall, dstall, krows0, krows1, qrows0, qrows1,
                  vrows0, vrows1, msg, zbuf, acc, sem0, sem1):
    cid = lax.axis_index("c")
    sid = lax.axis_index("s")
    w = cid * _NS + sid
    kbufs = (krows0, krows1)
    qbufs = (qrows0, qrows1)
    vbufs = (vrows0, vrows1)
    sems = (sem0, sem1)

    def zrow(j, carry):
      for f0 in range(F // 16):
        zbuf[j, pl.ds(f0 * 16, 16)] = jnp.zeros((16,), jnp.float32)
      return carry

    lax.fori_loop(0, _ZR, zrow, 0)
    for t in range(_RPT // _ZR):
      pltpu.sync_copy(zbuf, acc.at[pl.ds(sid * _RPT + t * _ZR, _ZR)])
    plsc.subcore_barrier()

    pltpu.sync_copy(src_hbm.at[pl.ds(w * _BASE, _BASE)], srcall)
    pltpu.sync_copy(dst_hbm.at[pl.ds(w * _BASE, _BASE)], dstall)

    def start_gathers(j, b):
      pltpu.async_copy(k_hbm.at[dstall.at[j]], kbufs[b], sems[b])
      pltpu.async_copy(q_hbm.at[srcall.at[j]], qbufs[b], sems[b])
      pltpu.async_copy(v_hbm.at[srcall.at[j]], vbufs[b], sems[b])

    def wait_gathers(j, b):
      pltpu.make_async_copy(k_hbm.at[dstall.at[j]], kbufs[b], sems[b]).wait()
      pltpu.make_async_copy(q_hbm.at[srcall.at[j]], qbufs[b], sems[b]).wait()
      pltpu.make_async_copy(v_hbm.at[srcall.at[j]], vbufs[b], sems[b]).wait()

    def compute_scatter(j, b):
      kb = kbufs[b]
      qb = qbufs[b]
      vb = vbufs[b]

      @plsc.parallel_loop(0, _B, unroll=8)
      def row(r):
        for g in range(F // 32):
          kq = kb[r, pl.ds(g * 32, 32)]
          qq = qb[r, pl.ds(g * 32, 32)]
          ka, kb2 = plsc.unpack(kq, format=plsc.PackFormat.INTERLEAVED,
                                preferred_element_type=jnp.float32)
          qa, qb2 = plsc.unpack(qq, format=plsc.PackFormat.INTERLEAVED,
                                preferred_element_type=jnp.float32)
          ga = 1.0 / (1.0 + jnp.exp(-(ka + qa)))
          gb = 1.0 / (1.0 + jnp.exp(-(kb2 + qb2)))
          msg[r, pl.ds(g * 32, 16)] = ga * vb[r, pl.ds(g * 32, 16)]
          msg[r, pl.ds(g * 32 + 16, 16)] = gb * vb[r, pl.ds(g * 32 + 16, 16)]

      pltpu.sync_copy(msg, acc.at[dstall.at[j]], add=True)

    start_gathers(0, 0)

    def body(j2, carry):
      for b in range(2):
        j = 2 * j2 + b

        @pl.when(j + 1 < _BASE)
        def _():
          start_gathers(j + 1, 1 - b)

        wait_gathers(j, b)
        compute_scatter(j, b)
      return carry

    lax.fori_loop(0, _BASE // 2, body, 0)

    @pl.when(w < _EXTRA)
    def _():
      pltpu.sync_copy(src_hbm.at[pl.ds(_NW * _BASE + w, 1)],
                      srcall.at[pl.ds(0, 1)])
      pltpu.sync_copy(dst_hbm.at[pl.ds(_NW * _BASE + w, 1)],
                      dstall.at[pl.ds(0, 1)])
      start_gathers(0, 0)
      wait_gathers(0, 0)
      compute_scatter(0, 0)

    plsc.subcore_barrier()
    pltpu.sync_copy(acc.at[pl.ds(sid * _RPT, _RPT)],
                    out_hbm.at[pl.ds(cid * _NP + sid * _RPT, _RPT)])

  return edge_kernel


_PERM64 = tuple(32 * g + (j // 2) + 16 * (j % 2)
                for g in range(2) for j in range(32))


def kernel(x, adj, We0, be0, ge0, bbe0, We1, be1, ge1, bbe1,
           Wk1, bk1, Wq1, bq1, Wv1, bv1, Ws1, bs1, gc1, bc1,
           Wkm, bkm, Wqm, bqm, Wvm, bvm, Wsm, bsm,
           Wkl, bkl, Wql, bql, Wvl, bvl, Wsl, bsl,
           Wd0, bd0, gd0, bbd0, Wd1, bd1, gd1, bbd1,
           Wd2, bd2, gd2, bbd2, cluster):
  src2d = adj[0].reshape(_CHUNKS, _B)
  dst2d = adj[1].reshape(_CHUNKS, _B)

  Wqv1 = jnp.concatenate([Wq1, Wv1], axis=1)
  bqv1 = jnp.concatenate([bq1, bv1], axis=0)
  Wk2 = jnp.concatenate([Wkm, Wkl], axis=1)
  bk2 = jnp.concatenate([bkm, bkl], axis=0)
  Wqv2 = jnp.concatenate([Wqm, Wql, Wvm, Wvl], axis=1)
  bqv2 = jnp.concatenate([bqm, bql, bvm, bvl], axis=0)
  Ws2 = jnp.concatenate([Wsm, Wsl], axis=1)
  bs2 = jnp.concatenate([bsm, bsl], axis=0)

  feat_x, k1, qv1, s1 = pl.pallas_call(
      _tc1_body,
      grid=(_GRID,),
      in_specs=[_row_spec(128)] + [_full_spec(a) for a in
                (We0, be0, ge0, bbe0, We1, be1, ge1, bbe1,
                 Wk1, bk1, Wqv1, bqv1, Ws1, bs1)],
      out_specs=[_row_spec(20), _row_spec(64), _row_spec(128), _row_spec(64)],
      out_shape=[
          jax.ShapeDtypeStruct((_N, 20), jnp.float32),
          jax.ShapeDtypeStruct((_N, 64), jnp.float32),
          jax.ShapeDtypeStruct((_N, 128), jnp.float32),
          jax.ShapeDtypeStruct((_N, 64), jnp.float32),
      ],
  )(x, We0, be0, ge0, bbe0, We1, be1, ge1, bbe1, Wk1, bk1, Wqv1, bqv1, Ws1, bs1)

  perm = jnp.asarray(_PERM64, dtype=jnp.int32)
  k1bf = k1[:, perm].astype(jnp.bfloat16)
  q1bf = qv1[:, :64][:, perm].astype(jnp.bfloat16)
  v1 = qv1[:, 64:]
  agg1 = _make_edge_kernel_bf16_64()(k1bf, q1bf, v1, src2d, dst2d)

  k2, qv2, s2 = pl.pallas_call(
      _tc2_body,
      grid=(_GRID,),
      in_specs=[_row_spec(64), _row_spec(64), _row_spec(64)] +
               [_full_spec(a) for a in (gc1, bc1, Wk2, bk2, Wqv2, bqv2,
                                        Ws2, bs2)],
      out_specs=[_row_spec(16), _row_spec(32), _row_spec(16)],
      out_shape=[
          jax.ShapeDtypeStruct((_N, 16), jnp.float32),
          jax.ShapeDtypeStruct((_N, 32), jnp.float32),
          jax.ShapeDtypeStruct((_N, 16), jnp.float32),
      ],
  )(agg1[:_N], agg1[_NP:_NP + _N], s1, gc1, bc1, Wk2, bk2, Wqv2, bqv2, Ws2, bs2)

  agg2 = _make_edge_kernel(16, 32)(k2, qv2, src2d, dst2d)

  z, mu, logvar, de_feat, q = pl.pallas_call(
      _tc3_body,
      grid=(_GRID,),
      in_specs=[_row_spec(20), _row_spec(16), _row_spec(16), _row_spec(16)] +
               [_full_spec(a) for a in (Wd0, bd0, gd0, bbd0, Wd1, bd1,
                                        gd1, bbd1, Wd2, bd2, gd2, bbd2,
                                        cluster)],
      out_specs=[_row_spec(28), _row_spec(8), _row_spec(8), _row_spec(128),
                 _row_spec(15)],
      out_shape=[
          jax.ShapeDtypeStruct((_N, 28), jnp.float32),
          jax.ShapeDtypeStruct((_N, 8), jnp.float32),
          jax.ShapeDtypeStruct((_N, 8), jnp.float32),
          jax.ShapeDtypeStruct((_N, 128), jnp.float32),
          jax.ShapeDtypeStruct((_N, 15), jnp.float32),
      ],
  )(feat_x, agg2[:_N], agg2[_NP:_NP + _N], s2, Wd0, bd0, gd0, bbd0,
    Wd1, bd1, gd1, bbd1, Wd2, bd2, gd2, bbd2, cluster)

  return (z, mu, logvar, de_feat, q, feat_x, mu)


# speedup vs baseline: 1.0263x; 1.0263x over previous
import functools

import jax
import jax.numpy as jnp
from jax import lax
from jax.experimental import pallas as pl
from jax.experimental.pallas import tpu as pltpu
from jax.experimental.pallas import tpu_sc as plsc

_N = 10000
_E = 320000
_ALPHA = 0.8
_BN_EPS = 0.001
_PYG_BN_EPS = 1e-05


_BLK = 2000
_GRID = _N // _BLK

_P = jax.lax.Precision.HIGHEST


def _full_spec(arr):
  nd = arr.ndim
  return pl.BlockSpec(arr.shape, lambda i, _nd=nd: (0,) * _nd)


def _row_spec(cols):
  return pl.BlockSpec((_BLK, cols), lambda i: (i, 0))


def _elu(x):
  return jnp.where(x > 0, x, jnp.exp(x) - 1.0)


def _sigmoid(x):
  return 1.0 / (1.0 + jnp.exp(-x))


def _bn(x, g, b, eps):
  return x / jnp.sqrt(1.0 + eps) * g[None, :] + b[None, :]


def _tc1_body(x_ref, We0_ref, be0_ref, ge0_ref, bbe0_ref,
              We1_ref, be1_ref, ge1_ref, bbe1_ref,
              Wk_ref, bk_ref, Wq_ref, bq_ref, Wv_ref, bv_ref,
              Ws_ref, bs_ref,
              f_ref, k_ref, qv_ref, s_ref):
  x = x_ref[...]
  h = _elu(_bn(x @ We0_ref[...] + be0_ref[...][None, :],
               ge0_ref[...], bbe0_ref[...], _BN_EPS))
  f = _elu(_bn(h @ We1_ref[...] + be1_ref[...][None, :],
               ge1_ref[...], bbe1_ref[...], _BN_EPS))
  f_ref[...] = f
  k_ref[...] = f @ Wk_ref[...] + bk_ref[...][None, :]
  qv_ref[:, :64] = f @ Wq_ref[...] + bq_ref[...][None, :]
  qv_ref[:, 64:] = f @ Wv_ref[...] + bv_ref[...][None, :]
  s_ref[...] = f @ Ws_ref[...] + bs_ref[...][None, :]


def _tc2_body(a0_ref, a1_ref, s1_ref, gc_ref, bc_ref,
              Wkm_ref, bkm_ref, Wkl_ref, bkl_ref,
              Wqm_ref, bqm_ref, Wql_ref, bql_ref,
              Wvm_ref, bvm_ref, Wvl_ref, bvl_ref,
              Wsm_ref, bsm_ref, Wsl_ref, bsl_ref,
              k_ref, qv_ref, s_ref):
  c1 = a0_ref[...] + a1_ref[...] + s1_ref[...]
  c1 = jnp.maximum(_bn(c1, gc_ref[...], bc_ref[...], _PYG_BN_EPS), 0.0)
  k_ref[:, :8] = c1 @ Wkm_ref[...] + bkm_ref[...][None, :]
  k_ref[:, 8:] = c1 @ Wkl_ref[...] + bkl_ref[...][None, :]
  qv_ref[:, :8] = c1 @ Wqm_ref[...] + bqm_ref[...][None, :]
  qv_ref[:, 8:16] = c1 @ Wql_ref[...] + bql_ref[...][None, :]
  qv_ref[:, 16:24] = c1 @ Wvm_ref[...] + bvm_ref[...][None, :]
  qv_ref[:, 24:] = c1 @ Wvl_ref[...] + bvl_ref[...][None, :]
  s_ref[:, :8] = c1 @ Wsm_ref[...] + bsm_ref[...][None, :]
  s_ref[:, 8:] = c1 @ Wsl_ref[...] + bsl_ref[...][None, :]


def _tc3_body(f_ref, a0_ref, a1_ref, s2_ref,
              Wd0_ref, bd0_ref, gd0_ref, bbd0_ref,
              Wd1_ref, bd1_ref, gd1_ref, bbd1_ref,
              Wd2_ref, bd2_ref, gd2_ref, bbd2_ref,
              cl_ref,
              z_ref, mu_ref, lv_ref, de_ref, q_ref):
  t = a0_ref[...] + a1_ref[...] + s2_ref[...]
  mu = t[:, :8]
  lv = t[:, 8:]
  mu_ref[...] = mu
  lv_ref[...] = lv
  z = jnp.concatenate([f_ref[...], mu], axis=1)
  z_ref[...] = z
  d = _elu(_bn(z @ Wd0_ref[...] + bd0_ref[...][None, :],
               gd0_ref[...], bbd0_ref[...], _BN_EPS))
  d = _elu(_bn(d @ Wd1_ref[...] + bd1_ref[...][None, :],
               gd1_ref[...], bbd1_ref[...], _BN_EPS))
  de_ref[...] = _sigmoid(_bn(d @ Wd2_ref[...] + bd2_ref[...][None, :],
                             gd2_ref[...], bbd2_ref[...], _BN_EPS))
  cl = cl_ref[...]
  zn2 = jnp.sum(z * z, axis=1, keepdims=True)
  cn2 = jnp.sum(cl * cl, axis=1)[None, :]
  cross = jax.lax.dot_general(z, cl, (((1,), (1,)), ((), ())))
  dist2 = zn2 - 2.0 * cross + cn2
  base = 1.0 + dist2 / _ALPHA + 1e-08
  qq = jnp.exp((-(_ALPHA + 1.0) / 2.0) * jnp.log(base))
  q_ref[...] = qq / jnp.sum(qq, axis=1, keepdims=True)


_NC = 2
_NS = 16
_NW = _NC * _NS
_B = 128
_CHUNKS = _E // _B
_BASE = _CHUNKS // _NW
_EXTRA = _CHUNKS - _BASE * _NW
_NP = 10240
_RPT = _NP // _NS
_ZR = 128


@functools.lru_cache(maxsize=None)
def _make_edge_kernel(F, FQV):
  mesh = plsc.VectorSubcoreMesh(core_axis_name="c", subcore_axis_name="s",
                                num_cores=_NC, num_subcores=_NS)
  nf = F // 16

  @functools.partial(
      pl.kernel,
      out_type=jax.ShapeDtypeStruct((_NC * _NP, F), jnp.float32),
      mesh=mesh,
      compiler_params=pltpu.CompilerParams(use_tc_tiling_on_sc=False),
      scratch_types=[
          pltpu.VMEM((_BASE, _B), jnp.int32),
          pltpu.VMEM((_BASE, _B), jnp.int32),
          pltpu.VMEM((_B, F), jnp.float32),
          pltpu.VMEM((_B, F), jnp.float32),
          pltpu.VMEM((_B, FQV), jnp.float32),
          pltpu.VMEM((_B, FQV), jnp.float32),
          pltpu.VMEM((_B, F), jnp.float32),
          pltpu.VMEM((_ZR, F), jnp.float32),
          pltpu.VMEM_SHARED((_NP, F), jnp.float32),
          pltpu.SemaphoreType.DMA,
          pltpu.SemaphoreType.DMA,
      ],
  )
  def edge_kernel(k_hbm, qv_hbm, src_hbm, dst_hbm, out_hbm,
                  srcall, dstall, krows0, krows1, qvrows0, qvrows1,
                  msg, zbuf, acc, sem0, sem1):
    cid = lax.axis_index("c")
    sid = lax.axis_index("s")
    w = cid * _NS + sid
    kbufs = (krows0, krows1)
    qbufs = (qvrows0, qvrows1)
    sems = (sem0, sem1)

    def zrow(j, carry):
      for f0 in range(nf):
        zbuf[j, pl.ds(f0 * 16, 16)] = jnp.zeros((16,), jnp.float32)
      return carry

    lax.fori_loop(0, _ZR, zrow, 0)
    for t in range(_RPT // _ZR):
      pltpu.sync_copy(zbuf, acc.at[pl.ds(sid * _RPT + t * _ZR, _ZR)])
    plsc.subcore_barrier()

    pltpu.sync_copy(src_hbm.at[pl.ds(w * _BASE, _BASE)], srcall)
    pltpu.sync_copy(dst_hbm.at[pl.ds(w * _BASE, _BASE)], dstall)

    def start_gathers(j, b):
      pltpu.async_copy(k_hbm.at[dstall.at[j]], kbufs[b], sems[b])
      pltpu.async_copy(qv_hbm.at[srcall.at[j]], qbufs[b], sems[b])

    def wait_gathers(j, b):
      pltpu.make_async_copy(k_hbm.at[dstall.at[j]], kbufs[b], sems[b]).wait()
      pltpu.make_async_copy(qv_hbm.at[srcall.at[j]], qbufs[b], sems[b]).wait()

    def compute_scatter(j, b):
      kb = kbufs[b]
      qb = qbufs[b]

      @plsc.parallel_loop(0, _B, unroll=8)
      def row(r):
        for f0 in range(nf):
          kv = kb[r, pl.ds(f0 * 16, 16)]
          qv = qb[r, pl.ds(f0 * 16, 16)]
          vv = qb[r, pl.ds(F + f0 * 16, 16)]
          gate = 1.0 / (1.0 + jnp.exp(-(kv + qv)))
          msg[r, pl.ds(f0 * 16, 16)] = gate * vv

      pltpu.sync_copy(msg, acc.at[dstall.at[j]], add=True)

    start_gathers(0, 0)

    def body(j2, carry):
      for b in range(2):
        j = 2 * j2 + b

        @pl.when(j + 1 < _BASE)
        def _():
          start_gathers(j + 1, 1 - b)

        wait_gathers(j, b)
        compute_scatter(j, b)
      return carry

    lax.fori_loop(0, _BASE // 2, body, 0)

    @pl.when(w < _EXTRA)
    def _():
      pltpu.sync_copy(src_hbm.at[pl.ds(_NW * _BASE + w, 1)],
                      srcall.at[pl.ds(0, 1)])
      pltpu.sync_copy(dst_hbm.at[pl.ds(_NW * _BASE + w, 1)],
                      dstall.at[pl.ds(0, 1)])
      start_gathers(0, 0)
      wait_gathers(0, 0)
      compute_scatter(0, 0)

    plsc.subcore_barrier()
    pltpu.sync_copy(acc.at[pl.ds(sid * _RPT, _RPT)],
                    out_hbm.at[pl.ds(cid * _NP + sid * _RPT, _RPT)])

  return edge_kernel


def kernel(x, adj, We0, be0, ge0, bbe0, We1, be1, ge1, bbe1,
           Wk1, bk1, Wq1, bq1, Wv1, bv1, Ws1, bs1, gc1, bc1,
           Wkm, bkm, Wqm, bqm, Wvm, bvm, Wsm, bsm,
           Wkl, bkl, Wql, bql, Wvl, bvl, Wsl, bsl,
           Wd0, bd0, gd0, bbd0, Wd1, bd1, gd1, bbd1,
           Wd2, bd2, gd2, bbd2, cluster):
  src2d = adj[0].reshape(_CHUNKS, _B)
  dst2d = adj[1].reshape(_CHUNKS, _B)

  feat_x, k1, qv1, s1 = pl.pallas_call(
      _tc1_body,
      grid=(_GRID,),
      in_specs=[_row_spec(128)] + [_full_spec(a) for a in
                (We0, be0, ge0, bbe0, We1, be1, ge1, bbe1,
                 Wk1, bk1, Wq1, bq1, Wv1, bv1, Ws1, bs1)],
      out_specs=[_row_spec(20), _row_spec(64), _row_spec(128), _row_spec(64)],
      out_shape=[
          jax.ShapeDtypeStruct((_N, 20), jnp.float32),
          jax.ShapeDtypeStruct((_N, 64), jnp.float32),
          jax.ShapeDtypeStruct((_N, 128), jnp.float32),
          jax.ShapeDtypeStruct((_N, 64), jnp.float32),
      ],
  )(x, We0, be0, ge0, bbe0, We1, be1, ge1, bbe1, Wk1, bk1, Wq1, bq1, Wv1, bv1, Ws1, bs1)

  agg1 = _make_edge_kernel(64, 128)(k1, qv1, src2d, dst2d)

  k2, qv2, s2 = pl.pallas_call(
      _tc2_body,
      grid=(_GRID,),
      in_specs=[_row_spec(64), _row_spec(64), _row_spec(64)] +
               [_full_spec(a) for a in (gc1, bc1, Wkm, bkm, Wkl, bkl,
                                        Wqm, bqm, Wql, bql, Wvm, bvm,
                                        Wvl, bvl, Wsm, bsm, Wsl, bsl)],
      out_specs=[_row_spec(16), _row_spec(32), _row_spec(16)],
      out_shape=[
          jax.ShapeDtypeStruct((_N, 16), jnp.float32),
          jax.ShapeDtypeStruct((_N, 32), jnp.float32),
          jax.ShapeDtypeStruct((_N, 16), jnp.float32),
      ],
  )(agg1[:_N], agg1[_NP:_NP + _N], s1, gc1, bc1, Wkm, bkm, Wkl, bkl, Wqm, bqm, Wql, bql, Wvm, bvm, Wvl, bvl, Wsm, bsm, Wsl, bsl)

  agg2 = _make_edge_kernel(16, 32)(k2, qv2, src2d, dst2d)

  z, mu, logvar, de_feat, q = pl.pallas_call(
      _tc3_body,
      grid=(_GRID,),
      in_specs=[_row_spec(20), _row_spec(16), _row_spec(16), _row_spec(16)] +
               [_full_spec(a) for a in (Wd0, bd0, gd0, bbd0, Wd1, bd1,
                                        gd1, bbd1, Wd2, bd2, gd2, bbd2,
                                        cluster)],
      out_specs=[_row_spec(28), _row_spec(8), _row_spec(8), _row_spec(128),
                 _row_spec(15)],
      out_shape=[
          jax.ShapeDtypeStruct((_N, 28), jnp.float32),
          jax.ShapeDtypeStruct((_N, 8), jnp.float32),
          jax.ShapeDtypeStruct((_N, 8), jnp.float32),
          jax.ShapeDtypeStruct((_N, 128), jnp.float32),
          jax.ShapeDtypeStruct((_N, 15), jnp.float32),
      ],
  )(feat_x, agg2[:_N], agg2[_NP:_NP + _N], s2, Wd0, bd0, gd0, bbd0,
    Wd1, bd1, gd1, bbd1, Wd2, bd2, gd2, bbd2, cluster)

  return (z, mu, logvar, de_feat, q, feat_x, mu)
